# baseline (speedup 1.0000x reference)
import functools

import jax
import jax.numpy as jnp
from jax import lax
from jax.experimental import pallas as pl
from jax.experimental.pallas import tpu as pltpu
from jax.experimental.pallas import tpu_sc as plsc

RANK = 64
GW = 128
BLK = 512
NP_PAD = 1024


def _sc_gather_t(tab1d, s_idx, o_idx, ne):
    b = s_idx.shape[0] * s_idx.shape[1]
    mesh = plsc.VectorSubcoreMesh(core_axis_name="core", subcore_axis_name="subcore")
    out_t = (
        jax.ShapeDtypeStruct((RANK, b), jnp.float32),
        jax.ShapeDtypeStruct((RANK, b), jnp.float32),
    )

    @functools.partial(
        pl.kernel, out_type=out_t, mesh=mesh,
        compiler_params=pltpu.CompilerParams(use_tc_tiling_on_sc=False),
        scratch_types=[
            pltpu.VMEM((RANK, GW), jnp.int32),
            pltpu.SemaphoreType.DMA,
        ],
    )
    def gather_kernel(tab_hbm, si_hbm, oi_hbm, st_hbm, ot_hbm, g2, sem):
        def one_table(idx_v, out_v):
            dummy = st_hbm.at[:, pl.ds(0, GW)]
            @pl.loop(0, RANK)
            def _(d):
                off = d * ne
                for k in range(GW // 16):
                    sl = pl.ds(16 * k, 16)
                    g2[d, sl] = idx_v[0, sl] + off
            @pl.loop(0, RANK)
            def _(d):
                pltpu.make_async_copy(tab_hbm.at[g2.at[d]], out_v.at[d], sem).start()
            pltpu.make_async_copy(dummy, out_v, sem).wait()

        def body(si_v, oi_v, st_v, ot_v):
            one_table(si_v, st_v)
            one_table(oi_v, ot_v)

        pltpu.emit_pipeline(
            body,
            grid=(b // GW,),
            in_specs=[
                pl.BlockSpec((1, GW), lambda i: (i, 0)),
                pl.BlockSpec((1, GW), lambda i: (i, 0)),
            ],
            out_specs=[
                pl.BlockSpec((RANK, GW), lambda i: (0, i)),
                pl.BlockSpec((RANK, GW), lambda i: (0, i)),
            ],
            core_axis_name=("core", "subcore"),
            dimension_semantics=(pltpu.PARALLEL,),
        )(si_hbm, oi_hbm, st_hbm, ot_hbm)

    return gather_kernel(tab1d, s_idx, o_idx)


def _tc_body(st_ref, ot_ref, p_ref, pt_ref, smt_ref, tmt_ref, out_ref):
    st = st_ref[...].astype(jnp.bfloat16)
    ot = ot_ref[...].astype(jnp.bfloat16)
    s_rep = lax.dot_general(smt_ref[...], st, (((1,), (0,)), ((), ())),
                            preferred_element_type=jnp.float32)
    o_tile = lax.dot_general(tmt_ref[...], ot, (((1,), (0,)), ((), ())),
                             preferred_element_type=jnp.float32)
    xt = (s_rep * o_tile).astype(jnp.bfloat16)
    scores = lax.dot_general(pt_ref[...], xt, (((1,), (0,)), ((), ())),
                             preferred_element_type=jnp.float32)
    pidx = p_ref[0]
    sel = pidx == lax.broadcasted_iota(jnp.int32, (NP_PAD, BLK), 0)
    spo = jnp.sum(jnp.where(sel, scores, 0.0), axis=0, keepdims=True)
    out_ref[...] = jax.nn.sigmoid(spo)[None]


def kernel(s_input, p_input, o_input, entity_table, predicate_table):
    b = s_input.shape[0]
    ne = entity_table.shape[0]
    np_real = predicate_table.shape[0]
    s_idx = s_input.reshape(b // GW, GW).astype(jnp.int32)
    o_idx = o_input.reshape(b // GW, GW).astype(jnp.int32)
    p3 = p_input.reshape(b // BLK, 1, BLK).astype(jnp.int32)

    tab1d = entity_table.T.reshape(ne * RANK)
    st, ot = _sc_gather_t(tab1d, s_idx, o_idx, ne)

    pflat = predicate_table.reshape(np_real, RANK * RANK).astype(jnp.bfloat16)
    pflat = jnp.pad(pflat, ((0, NP_PAD - np_real), (0, 0)))

    k = jnp.arange(RANK * RANK, dtype=jnp.int32)
    ar = jnp.arange(RANK, dtype=jnp.int32)
    smt = (k[:, None] // RANK == ar[None, :]).astype(jnp.bfloat16)
    tmt = (k[:, None] % RANK == ar[None, :]).astype(jnp.bfloat16)

    out = pl.pallas_call(
        _tc_body,
        grid=(b // BLK,),
        in_specs=[
            pl.BlockSpec((RANK, BLK), lambda i: (0, i)),
            pl.BlockSpec((RANK, BLK), lambda i: (0, i)),
            pl.BlockSpec((1, 1, BLK), lambda i: (i, 0, 0)),
            pl.BlockSpec((NP_PAD, RANK * RANK), lambda i: (0, 0)),
            pl.BlockSpec((RANK * RANK, RANK), lambda i: (0, 0)),
            pl.BlockSpec((RANK * RANK, RANK), lambda i: (0, 0)),
        ],
        out_specs=pl.BlockSpec((1, 1, BLK), lambda i: (i, 0, 0)),
        out_shape=jax.ShapeDtypeStruct((b // BLK, 1, BLK), jnp.float32),
    )(st, ot, p3, pflat, smt, tmt)
    return out.reshape(b, 1)


# speedup vs baseline: 7.7147x; 7.7147x over previous
import jax
import jax.numpy as jnp
from jax import lax
from jax.experimental import pallas as pl
from jax.experimental.pallas import tpu as pltpu

RANK = 64
BLK = 1024
NP_PAD = 1024


def _issue(tab_ref, idx_ref, base, buf, slot, sem):
    def one(j, _):
        idx = idx_ref[base + j]
        pltpu.make_async_copy(
            tab_ref.at[pl.ds(idx, 1)],
            buf.at[slot, pl.ds(j, 1)],
            sem.at[slot],
        ).start()
        return 0

    lax.fori_loop(0, BLK, one, 0, unroll=8)


def _wait(buf, slot, sem):
    pltpu.make_async_copy(buf.at[slot], buf.at[slot], sem.at[slot]).wait()


def _body(si_ref, oi_ref, tab_ref, p_ref, pt_ref, sm_ref, tm_ref, out_ref,
          sbuf, obuf, sem_s, sem_o):
    i = pl.program_id(0)
    n = pl.num_programs(0)
    slot = lax.rem(i, 2)

    @pl.when(i == 0)
    def _prologue():
        _issue(tab_ref, si_ref, 0, sbuf, 0, sem_s)
        _issue(tab_ref, oi_ref, 0, obuf, 0, sem_o)

    @pl.when(i + 1 < n)
    def _prefetch_next():
        nxt = lax.rem(i + 1, 2)
        _issue(tab_ref, si_ref, (i + 1) * BLK, sbuf, nxt, sem_s)
        _issue(tab_ref, oi_ref, (i + 1) * BLK, obuf, nxt, sem_o)

    _wait(sbuf, slot, sem_s)
    _wait(obuf, slot, sem_o)

    s = sbuf[slot].astype(jnp.bfloat16)
    o = obuf[slot].astype(jnp.bfloat16)
    s_rep = lax.dot_general(s, sm_ref[...], (((1,), (0,)), ((), ())),
                            preferred_element_type=jnp.float32)
    o_tile = lax.dot_general(o, tm_ref[...], (((1,), (0,)), ((), ())),
                             preferred_element_type=jnp.float32)
    x = (s_rep * o_tile).astype(jnp.bfloat16)
    scores = lax.dot_general(x, pt_ref[...], (((1,), (0,)), ((), ())),
                             preferred_element_type=jnp.float32)
    pidx = p_ref[0]
    sel = pidx == lax.broadcasted_iota(jnp.int32, (BLK, NP_PAD), 1)
    spo = jnp.sum(jnp.where(sel, scores, 0.0), axis=1, keepdims=True)
    out_ref[...] = jax.nn.sigmoid(spo)


def kernel(s_input, p_input, o_input, entity_table, predicate_table):
    b = s_input.shape[0]
    np_real = predicate_table.shape[0]
    s_idx = s_input.reshape(b).astype(jnp.int32)
    o_idx = o_input.reshape(b).astype(jnp.int32)
    p3 = p_input.reshape(b // BLK, BLK, 1).astype(jnp.int32)

    ptt = predicate_table.reshape(np_real, RANK * RANK).astype(jnp.bfloat16)
    ptt = jnp.pad(ptt, ((0, NP_PAD - np_real), (0, 0))).T

    k = jnp.arange(RANK * RANK, dtype=jnp.int32)
    ar = jnp.arange(RANK, dtype=jnp.int32)
    sm = (ar[:, None] == k[None, :] // RANK).astype(jnp.bfloat16)
    tm = (ar[:, None] == k[None, :] % RANK).astype(jnp.bfloat16)

    grid_spec = pltpu.PrefetchScalarGridSpec(
        num_scalar_prefetch=2,
        grid=(b // BLK,),
        in_specs=[
            pl.BlockSpec(memory_space=pl.ANY),
            pl.BlockSpec((1, BLK, 1), lambda i, si, oi: (i, 0, 0)),
            pl.BlockSpec((RANK * RANK, NP_PAD), lambda i, si, oi: (0, 0)),
            pl.BlockSpec((RANK, RANK * RANK), lambda i, si, oi: (0, 0)),
            pl.BlockSpec((RANK, RANK * RANK), lambda i, si, oi: (0, 0)),
        ],
        out_specs=pl.BlockSpec((BLK, 1), lambda i, si, oi: (i, 0)),
        scratch_shapes=[
            pltpu.VMEM((2, BLK, RANK), jnp.float32),
            pltpu.VMEM((2, BLK, RANK), jnp.float32),
            pltpu.SemaphoreType.DMA((2,)),
            pltpu.SemaphoreType.DMA((2,)),
        ],
    )
    out = pl.pallas_call(
        _body,
        grid_spec=grid_spec,
        out_shape=jax.ShapeDtypeStruct((b, 1), jnp.float32),
    )(s_idx, o_idx, entity_table, p3, ptt, sm, tm)
    return out


# speedup vs baseline: 7.9457x; 1.0300x over previous
import jax
import jax.numpy as jnp
from jax import lax
from jax.experimental import pallas as pl
from jax.experimental.pallas import tpu as pltpu

RANK = 64
BLK = 512
NP_PAD = 1024


def _issue(tab_ref, si_ref, oi_ref, base, sbuf, obuf, slot, sem_s, sem_o):
    def one(j, _):
        si = si_ref[base + j]
        oi = oi_ref[base + j]
        pltpu.make_async_copy(
            tab_ref.at[pl.ds(si, 1)],
            sbuf.at[slot, pl.ds(j, 1)],
            sem_s.at[slot],
        ).start()
        pltpu.make_async_copy(
            tab_ref.at[pl.ds(oi, 1)],
            obuf.at[slot, pl.ds(j, 1)],
            sem_o.at[slot],
        ).start()
        return 0

    lax.fori_loop(0, BLK, one, 0, unroll=8)


def _wait(buf, slot, sem):
    pltpu.make_async_copy(buf.at[slot], buf.at[slot], sem.at[slot]).wait()


def _body(si_ref, oi_ref, tab_ref, p_ref, pt_ref, sm_ref, tm_ref, out_ref,
          sbuf, obuf, sem_s, sem_o):
    i = pl.program_id(0)
    n = pl.num_programs(0)
    slot = lax.rem(i, 2)

    @pl.when(i == 0)
    def _prologue():
        _issue(tab_ref, si_ref, oi_ref, 0, sbuf, obuf, 0, sem_s, sem_o)

    @pl.when(i + 1 < n)
    def _prefetch_next():
        nxt = lax.rem(i + 1, 2)
        _issue(tab_ref, si_ref, oi_ref, (i + 1) * BLK, sbuf, obuf, nxt,
               sem_s, sem_o)

    _wait(sbuf, slot, sem_s)
    _wait(obuf, slot, sem_o)

    s = sbuf[slot].astype(jnp.bfloat16)
    o = obuf[slot].astype(jnp.bfloat16)
    s_rep = lax.dot_general(s, sm_ref[...], (((1,), (0,)), ((), ())),
                            preferred_element_type=jnp.float32)
    o_tile = lax.dot_general(o, tm_ref[...], (((1,), (0,)), ((), ())),
                             preferred_element_type=jnp.float32)
    x = (s_rep * o_tile).astype(jnp.bfloat16)
    scores = lax.dot_general(x, pt_ref[...], (((1,), (0,)), ((), ())),
                             preferred_element_type=jnp.float32)
    pidx = p_ref[0]
    sel = pidx == lax.broadcasted_iota(jnp.int32, (BLK, NP_PAD), 1)
    spo = jnp.sum(jnp.where(sel, scores, 0.0), axis=1, keepdims=True)
    out_ref[...] = jax.nn.sigmoid(spo)


def kernel(s_input, p_input, o_input, entity_table, predicate_table):
    b = s_input.shape[0]
    np_real = predicate_table.shape[0]
    s_idx = s_input.reshape(b).astype(jnp.int32)
    o_idx = o_input.reshape(b).astype(jnp.int32)
    p3 = p_input.reshape(b // BLK, BLK, 1).astype(jnp.int32)

    ptt = predicate_table.reshape(np_real, RANK * RANK).astype(jnp.bfloat16)
    ptt = jnp.pad(ptt, ((0, NP_PAD - np_real), (0, 0))).T

    k = jnp.arange(RANK * RANK, dtype=jnp.int32)
    ar = jnp.arange(RANK, dtype=jnp.int32)
    sm = (ar[:, None] == k[None, :] // RANK).astype(jnp.bfloat16)
    tm = (ar[:, None] == k[None, :] % RANK).astype(jnp.bfloat16)

    grid_spec = pltpu.PrefetchScalarGridSpec(
        num_scalar_prefetch=2,
        grid=(b // BLK,),
        in_specs=[
            pl.BlockSpec(memory_space=pl.ANY),
            pl.BlockSpec((1, BLK, 1), lambda i, si, oi: (i, 0, 0)),
            pl.BlockSpec((RANK * RANK, NP_PAD), lambda i, si, oi: (0, 0)),
            pl.BlockSpec((RANK, RANK * RANK), lambda i, si, oi: (0, 0)),
            pl.BlockSpec((RANK, RANK * RANK), lambda i, si, oi: (0, 0)),
        ],
        out_specs=pl.BlockSpec((BLK, 1), lambda i, si, oi: (i, 0)),
        scratch_shapes=[
            pltpu.VMEM((2, BLK, RANK), jnp.float32),
            pltpu.VMEM((2, BLK, RANK), jnp.float32),
            pltpu.SemaphoreType.DMA((2,)),
            pltpu.SemaphoreType.DMA((2,)),
        ],
    )
    out = pl.pallas_call(
        _body,
        grid_spec=grid_spec,
        out_shape=jax.ShapeDtypeStruct((b, 1), jnp.float32),
    )(s_idx, o_idx, entity_table, p3, ptt, sm, tm)
    return out
